# Initial kernel scaffold; baseline (speedup 1.0000x reference)
import jax
import jax.numpy as jnp
from jax.experimental import pallas as pl


def kernel(encoding_g, volume_mesh_centers, p_grid, W1, b1, W2, b2):
    raise NotImplementedError("write your pallas kernel here")


# speedup vs baseline: 50.8003x; 50.8003x over previous
import functools

import jax
import jax.numpy as jnp
from jax import lax
from jax.experimental import pallas as pl
from jax.experimental.pallas import tpu as pltpu
from jax.experimental.pallas import tpu_sc as plsc

RADIUS = 0.1
K = 64
C = 8
M = 32768
N = 16384
BASE = 512
G = 10
SLOT_S0 = 1016
CAP = 1024
R2 = RADIUS * RADIUS
NW = 32
QPW = N // NW
INF = float("inf")

_i32 = jnp.int32
_f32 = jnp.float32


def _iota16():
    return lax.iota(_i32, 16)


def _cell1(v):
    return jnp.minimum((v * G).astype(_i32), G - 1)


def _splat(x):
    return jnp.full((16,), x, _i32)


def _sload(ref, idx):
    return plsc.load_gather(ref, [_splat(idx)])[0]


def _sstore(ref, idx, val):
    mask0 = _iota16() == 0
    plsc.store_scatter(ref, [_splat(idx)], jnp.full((16,), val), mask=mask0)


def _bin_body(ptsf_hbm, sx_hbm, sy_hbm, sz_hbm, spid_hbm, starts_hbm,
              ptsblkf, cidv, hist, histsh, allhist, startsv, basev,
              destbuf, pidbuf, xbuf, ybuf, zbuf):
    t = lax.axis_index("s")
    npts = M // 16
    p0 = t * npts
    iota = _iota16()

    pltpu.sync_copy(ptsf_hbm.at[pl.ds(3 * p0, 3 * npts)], ptsblkf)

    zeros16 = jnp.zeros((16,), _i32)
    for g in range(64):
        hist[pl.ds(16 * g, 16)] = zeros16

    def p1(g, _):
        i3 = (16 * g + iota) * 3
        px = plsc.load_gather(ptsblkf, [i3])
        py = plsc.load_gather(ptsblkf, [i3 + 1])
        pz = plsc.load_gather(ptsblkf, [i3 + 2])
        cid16 = (_cell1(px) * G + _cell1(py)) * G + _cell1(pz)
        cidv[pl.ds(16 * g, 16)] = cid16
        for l in range(16):
            cq = cid16[l]
            h = _sload(hist, cq)
            _sstore(hist, cq, h + 1)
        return 0

    lax.fori_loop(0, npts // 16, p1, 0)

    pltpu.sync_copy(hist, histsh.at[t])
    plsc.subcore_barrier()
    pltpu.sync_copy(histsh, allhist)

    def p2(k, carry):
        tot = jnp.zeros((16,), _i32)
        prior = jnp.zeros((16,), _i32)
        for tt in range(16):
            v = allhist[tt, pl.ds(16 * k, 16)]
            prior = jnp.where(tt == t, tot, prior)
            tot = tot + v
        incl = plsc.cumsum(tot)
        excl = incl - tot
        startsv[pl.ds(16 * k, 16)] = excl + carry
        basev[pl.ds(16 * k, 16)] = excl + carry + prior
        return carry + jnp.max(incl)

    lax.fori_loop(0, 64, p2, jnp.int32(0))

    @pl.when(t == 0)
    def _():
        cid0 = cidv[pl.ds(0, 16)][0]
        _sstore(startsv, SLOT_S0, _sload(startsv, cid0))
        pltpu.sync_copy(startsv, starts_hbm)

    def p3(ch, _):
        b = ch * 128

        def serial(g, _):
            i3 = (b + 16 * g + iota) * 3
            xbuf[pl.ds(16 * g, 16)] = plsc.load_gather(ptsblkf, [i3])
            ybuf[pl.ds(16 * g, 16)] = plsc.load_gather(ptsblkf, [i3 + 1])
            zbuf[pl.ds(16 * g, 16)] = plsc.load_gather(ptsblkf, [i3 + 2])
            pidbuf[pl.ds(16 * g, 16)] = p0 + b + 16 * g + iota
            cid16 = cidv[pl.ds(b + 16 * g, 16)]
            for l in range(16):
                cq = cid16[l]
                d = _sload(basev, cq)
                _sstore(basev, cq, d + 1)
                _sstore(destbuf, 16 * g + l, d)
            return 0

        lax.fori_loop(0, 8, serial, 0)

        pltpu.sync_copy(xbuf, sx_hbm.at[destbuf])
        pltpu.sync_copy(ybuf, sy_hbm.at[destbuf])
        pltpu.sync_copy(zbuf, sz_hbm.at[destbuf])
        pltpu.sync_copy(pidbuf, spid_hbm.at[destbuf])
        return 0

    lax.fori_loop(0, 16, p3, 0)


def _run_bin(pts):
    mesh = plsc.VectorSubcoreMesh(core_axis_name="c", subcore_axis_name="s",
                                  num_cores=1, num_subcores=16)
    f = pl.kernel(
        _bin_body,
        mesh=mesh,
        compiler_params=pltpu.CompilerParams(needs_layout_passes=False, use_tc_tiling_on_sc=False),
        out_type=[
            jax.ShapeDtypeStruct((M,), _f32),
            jax.ShapeDtypeStruct((M,), _f32),
            jax.ShapeDtypeStruct((M,), _f32),
            jax.ShapeDtypeStruct((M,), _i32),
            jax.ShapeDtypeStruct((1024,), _i32),
        ],
        scratch_types=[
            pltpu.VMEM((3 * (M // 16),), _f32),
            pltpu.VMEM((M // 16,), _i32),
            pltpu.VMEM((1024,), _i32),
            pltpu.VMEM_SHARED((16, 1024), _i32),
            pltpu.VMEM((16, 1024), _i32),
            pltpu.VMEM((1024,), _i32),
            pltpu.VMEM((1024,), _i32),
            pltpu.VMEM((128,), _i32),
            pltpu.VMEM((128,), _i32),
            pltpu.VMEM((128,), _f32),
            pltpu.VMEM((128,), _f32),
            pltpu.VMEM((128,), _f32),
        ],
    )
    return f(pts.reshape(3 * M))


def _merge_once(R, kd, vd):
    (k0, v0, k1, v1, k2, v2, k3, v3) = R
    bk, bv = plsc.sort_key_val(kd, vd)
    out = []
    for rk, rv in ((k0, v0), (k1, v1), (k2, v2), (k3, v3)):
        rbk = lax.rev(bk, (0,))
        rbv = lax.rev(bv, (0,))
        take = rk <= rbk
        lk = jnp.where(take, rk, rbk)
        lv = jnp.where(take, rv, rbv)
        hk = jnp.where(take, rbk, rk)
        hv = jnp.where(take, rbv, rv)
        nk, nv = plsc.sort_key_val(lk, lv)
        bk, bv = plsc.sort_key_val(hk, hv)
        out.extend((nk, nv))
    return tuple(out)


def _select_body(sx_hbm, sy_hbm, sz_hbm, starts_hbm, qs_hbm, map_hbm,
                 sxv, syv, szv, startsv, qv, cd2, cidb, outst):
    wid = lax.axis_index("s") * 2 + lax.axis_index("c")
    iota = _iota16()
    r2 = jnp.float32(R2)

    pltpu.sync_copy(sx_hbm, sxv)
    pltpu.sync_copy(sy_hbm, syv)
    pltpu.sync_copy(sz_hbm, szv)
    pltpu.sync_copy(starts_hbm, startsv)
    pltpu.sync_copy(qs_hbm.at[pl.ds(wid * QPW * 3, QPW * 3)], qv)
    s0 = _sload(startsv, SLOT_S0)

    def per_query(q, _):
        vq = plsc.load_gather(qv, [3 * q + jnp.minimum(iota, 2)])
        qx = vq[0]
        qy = vq[1]
        qz = vq[2]
        cvec = _cell1(vq)
        cx = cvec[0]
        cy = cvec[1]
        cz = cvec[2]
        zlo = jnp.maximum(cz - 1, 0)
        zhi = jnp.minimum(cz + 1, G - 1)

        def over_x(x, cnt):
            def over_y(y, cnt):
                rowc = (x * G + y) * G
                sseg = _sload(startsv, rowc + zlo)
                eseg = _sload(startsv, rowc + zhi + 1)
                nch = (eseg - sseg + 15) // 16

                def chunk(k, nc):
                    lane = sseg + k * 16 + iota
                    lm = lane < eseg
                    li = jnp.minimum(lane, M - 1)
                    dx = qx - plsc.load_gather(sxv, [li])
                    dy = qy - plsc.load_gather(syv, [li])
                    dz = qz - plsc.load_gather(szv, [li])
                    d2 = (dx * dx + dy * dy) + dz * dz
                    w = lm & (d2 <= r2)
                    kd = jnp.where(w, d2, INF)
                    vd = jnp.where(w, lane, -1)
                    keep = (jnp.sum(w.astype(_i32)) > 0) & (nc < CAP // 16 - 1)
                    off = 16 * jnp.where(keep, nc, CAP // 16 - 1)
                    cd2[pl.ds(off, 16)] = kd
                    cidb[pl.ds(off, 16)] = vd
                    return nc + keep.astype(_i32)

                return lax.fori_loop(0, nch, chunk, cnt)

            return lax.fori_loop(jnp.maximum(cy - 1, 0),
                                 jnp.minimum(cy + 1, G - 1) + 1, over_y, cnt)

        cnt = lax.fori_loop(jnp.maximum(cx - 1, 0),
                            jnp.minimum(cx + 1, G - 1) + 1, over_x,
                            jnp.int32(0))

        inf16 = jnp.full((16,), INF, _f32)
        neg16 = jnp.full((16,), -1, _i32)
        R0 = (inf16, neg16, inf16, neg16, inf16, neg16, inf16, neg16)

        def sel(i, R):
            kd = cd2[pl.ds(16 * i, 16)]
            vd = cidb[pl.ds(16 * i, 16)]
            pred = jnp.min(kd) < jnp.max(R[6])
            return lax.cond(pred, lambda: _merge_once(R, kd, vd), lambda: R)

        R = lax.fori_loop(0, cnt, sel, R0)

        base = (q & 31) * 64
        for j in range(4):
            sid = R[2 * j + 1]
            res = jnp.where((sid == s0) | (sid < 0), M, sid)
            outst[pl.ds(base + 16 * j, 16)] = res

        @pl.when((q & 31) == 31)
        def _():
            pltpu.sync_copy(
                outst, map_hbm.at[pl.ds(wid * (QPW * K) + (q - 31) * 64, 2048)])

        return 0

    lax.fori_loop(0, QPW, per_query, 0)


def _run_select(sx, sy, sz, starts, qs_flat):
    mesh = plsc.VectorSubcoreMesh(core_axis_name="c", subcore_axis_name="s",
                                  num_cores=2, num_subcores=16)
    f = pl.kernel(
        _select_body,
        mesh=mesh,
        compiler_params=pltpu.CompilerParams(needs_layout_passes=False, use_tc_tiling_on_sc=False),
        out_type=jax.ShapeDtypeStruct((N * K,), _i32),
        scratch_types=[
            pltpu.VMEM((M,), _f32),
            pltpu.VMEM((M,), _f32),
            pltpu.VMEM((M,), _f32),
            pltpu.VMEM((1024,), _i32),
            pltpu.VMEM((QPW * 3,), _f32),
            pltpu.VMEM((CAP,), _f32),
            pltpu.VMEM((CAP,), _i32),
            pltpu.VMEM((2048,), _i32),
        ],
    )
    return f(sx, sy, sz, starts, qs_flat)


def _gather_body(encT_hbm, spid_hbm, map_hbm, feat_hbm, spidv, idxv, idx2, rows, sem):
    wid = lax.axis_index("s") * 2 + lax.axis_index("c")
    base = wid * (QPW * K)
    iota = _iota16()

    pltpu.sync_copy(spid_hbm, spidv)

    def chunk(ch, _):
        off = base + ch * 2048
        pltpu.sync_copy(map_hbm.at[pl.ds(off, 2048)], idxv)

        def xlat(g, _):
            sid = idxv[pl.ds(16 * g, 16)]
            pid = plsc.load_gather(spidv, [jnp.minimum(sid, M - 1)])
            idx2[pl.ds(16 * g, 16)] = jnp.where(sid == M, M, pid)
            return 0

        lax.fori_loop(0, 128, xlat, 0)

        cps = []
        for j in range(16):
            cps.append(pltpu.async_copy(
                encT_hbm.at[idx2.at[pl.ds(j * 128, 128)]],
                rows.at[pl.ds(j * 128, 128)], sem))
        for cp in cps:
            cp.wait()
        pltpu.sync_copy(rows, feat_hbm.at[pl.ds(off, 2048)])
        return 0

    lax.fori_loop(0, (QPW * K) // 2048, chunk, 0)


def _run_gather(encT, spid, mapv):
    mesh = plsc.VectorSubcoreMesh(core_axis_name="c", subcore_axis_name="s",
                                  num_cores=2, num_subcores=16)
    f = pl.kernel(
        _gather_body,
        mesh=mesh,
        compiler_params=pltpu.CompilerParams(needs_layout_passes=False, use_tc_tiling_on_sc=False),
        out_type=jax.ShapeDtypeStruct((N * K, 8), _f32),
        scratch_types=[
            pltpu.VMEM((M,), _i32),
            pltpu.VMEM((2048,), _i32),
            pltpu.VMEM((2048,), _i32),
            pltpu.VMEM((2048, 8), _f32),
            pltpu.SemaphoreType.DMA,
        ],
    )
    return f(encT, spid, mapv)


BLK_N = 256


def _mlp_body(x_ref, w1_ref, b1_ref, w2_ref, b2_ref, o_ref):
    x = x_ref[...]
    h = jnp.dot(x, w1_ref[...], preferred_element_type=jnp.float32) + b1_ref[...]
    h = jnp.maximum(h, 0.0)
    o_ref[...] = jnp.dot(h, w2_ref[...], preferred_element_type=jnp.float32) + b2_ref[...]


def _mlp(feat, W1p, b1, W2, b2):
    n = feat.shape[0]
    return pl.pallas_call(
        _mlp_body,
        grid=(n // BLK_N,),
        in_specs=[
            pl.BlockSpec((BLK_N, C * K), lambda i: (i, 0)),
            pl.BlockSpec((C * K, BASE), lambda i: (0, 0)),
            pl.BlockSpec((BASE,), lambda i: (0,)),
            pl.BlockSpec((BASE, K), lambda i: (0, 0)),
            pl.BlockSpec((K,), lambda i: (0,)),
        ],
        out_specs=pl.BlockSpec((BLK_N, K), lambda i: (i, 0)),
        out_shape=jax.ShapeDtypeStruct((n, K), jnp.float32),
    )(feat, W1p, b1, W2, b2)


def kernel(encoding_g, volume_mesh_centers, p_grid, W1, b1, W2, b2):
    assert volume_mesh_centers.shape[0] == 1
    pts = p_grid.reshape(M, 3)
    qs_flat = volume_mesh_centers.reshape(N * 3)
    encT = jnp.concatenate(
        [encoding_g.reshape(C, M).T, jnp.zeros((8, C), _f32)], axis=0)
    W1p = W1.reshape(C, K, BASE).transpose(1, 0, 2).reshape(K * C, BASE)

    sx, sy, sz, spid, starts = _run_bin(pts)
    mapv = _run_select(sx, sy, sz, starts, qs_flat)
    feat = _run_gather(encT, spid, mapv)
    out = _mlp(feat.reshape(N, K * C), W1p, b1, W2, b2)
    return out[None]


# speedup vs baseline: 76.7090x; 1.5100x over previous
import functools

import jax
import jax.numpy as jnp
from jax import lax
from jax.experimental import pallas as pl
from jax.experimental.pallas import tpu as pltpu
from jax.experimental.pallas import tpu_sc as plsc

RADIUS = 0.1
K = 64
C = 8
M = 32768
N = 16384
BASE = 512
G = 10
SLOT_S0 = 1016
CAP = 1024
R2 = RADIUS * RADIUS
NW = 32
QPW = N // NW
INF = float("inf")

_i32 = jnp.int32
_f32 = jnp.float32


def _iota16():
    return lax.iota(_i32, 16)


def _cell1(v):
    return jnp.minimum((v * G).astype(_i32), G - 1)


def _splat(x):
    return jnp.full((16,), x, _i32)


def _sload(ref, idx):
    return plsc.load_gather(ref, [_splat(idx)])[0]


def _sstore(ref, idx, val):
    mask0 = _iota16() == 0
    plsc.store_scatter(ref, [_splat(idx)], jnp.full((16,), val), mask=mask0)


def _bin_body(ptsf_hbm, sx_hbm, sy_hbm, sz_hbm, spid_hbm, starts_hbm,
              ptsblkf, cidv, hist, histsh, allhist, startsv, basev,
              destbuf, pidbuf, xbuf, ybuf, zbuf):
    t = lax.axis_index("s")
    npts = M // 16
    p0 = t * npts
    iota = _iota16()

    pltpu.sync_copy(ptsf_hbm.at[pl.ds(3 * p0, 3 * npts)], ptsblkf)

    zeros16 = jnp.zeros((16,), _i32)
    for g in range(64):
        hist[pl.ds(16 * g, 16)] = zeros16

    def p1(g, _):
        i3 = (16 * g + iota) * 3
        px = plsc.load_gather(ptsblkf, [i3])
        py = plsc.load_gather(ptsblkf, [i3 + 1])
        pz = plsc.load_gather(ptsblkf, [i3 + 2])
        cid16 = (_cell1(px) * G + _cell1(py)) * G + _cell1(pz)
        cidv[pl.ds(16 * g, 16)] = cid16
        for l in range(16):
            cq = cid16[l]
            h = _sload(hist, cq)
            _sstore(hist, cq, h + 1)
        return 0

    lax.fori_loop(0, npts // 16, p1, 0)

    pltpu.sync_copy(hist, histsh.at[t])
    plsc.subcore_barrier()
    pltpu.sync_copy(histsh, allhist)

    def p2(k, carry):
        tot = jnp.zeros((16,), _i32)
        prior = jnp.zeros((16,), _i32)
        for tt in range(16):
            v = allhist[tt, pl.ds(16 * k, 16)]
            prior = jnp.where(tt == t, tot, prior)
            tot = tot + v
        incl = plsc.cumsum(tot)
        excl = incl - tot
        startsv[pl.ds(16 * k, 16)] = excl + carry
        basev[pl.ds(16 * k, 16)] = excl + carry + prior
        return carry + jnp.max(incl)

    lax.fori_loop(0, 64, p2, jnp.int32(0))

    @pl.when(t == 0)
    def _():
        cid0 = cidv[pl.ds(0, 16)][0]
        _sstore(startsv, SLOT_S0, _sload(startsv, cid0))
        pltpu.sync_copy(startsv, starts_hbm)

    def p3(ch, _):
        b = ch * 128

        def serial(g, _):
            i3 = (b + 16 * g + iota) * 3
            xbuf[pl.ds(16 * g, 16)] = plsc.load_gather(ptsblkf, [i3])
            ybuf[pl.ds(16 * g, 16)] = plsc.load_gather(ptsblkf, [i3 + 1])
            zbuf[pl.ds(16 * g, 16)] = plsc.load_gather(ptsblkf, [i3 + 2])
            pidbuf[pl.ds(16 * g, 16)] = p0 + b + 16 * g + iota
            cid16 = cidv[pl.ds(b + 16 * g, 16)]
            for l in range(16):
                cq = cid16[l]
                d = _sload(basev, cq)
                _sstore(basev, cq, d + 1)
                _sstore(destbuf, 16 * g + l, d)
            return 0

        lax.fori_loop(0, 8, serial, 0)

        pltpu.sync_copy(xbuf, sx_hbm.at[destbuf])
        pltpu.sync_copy(ybuf, sy_hbm.at[destbuf])
        pltpu.sync_copy(zbuf, sz_hbm.at[destbuf])
        pltpu.sync_copy(pidbuf, spid_hbm.at[destbuf])
        return 0

    lax.fori_loop(0, 16, p3, 0)


def _run_bin(pts):
    mesh = plsc.VectorSubcoreMesh(core_axis_name="c", subcore_axis_name="s",
                                  num_cores=1, num_subcores=16)
    f = pl.kernel(
        _bin_body,
        mesh=mesh,
        compiler_params=pltpu.CompilerParams(needs_layout_passes=False, use_tc_tiling_on_sc=False),
        out_type=[
            jax.ShapeDtypeStruct((M,), _f32),
            jax.ShapeDtypeStruct((M,), _f32),
            jax.ShapeDtypeStruct((M,), _f32),
            jax.ShapeDtypeStruct((M,), _i32),
            jax.ShapeDtypeStruct((1024,), _i32),
        ],
        scratch_types=[
            pltpu.VMEM((3 * (M // 16),), _f32),
            pltpu.VMEM((M // 16,), _i32),
            pltpu.VMEM((1024,), _i32),
            pltpu.VMEM_SHARED((16, 1024), _i32),
            pltpu.VMEM((16, 1024), _i32),
            pltpu.VMEM((1024,), _i32),
            pltpu.VMEM((1024,), _i32),
            pltpu.VMEM((128,), _i32),
            pltpu.VMEM((128,), _i32),
            pltpu.VMEM((128,), _f32),
            pltpu.VMEM((128,), _f32),
            pltpu.VMEM((128,), _f32),
        ],
    )
    return f(pts.reshape(3 * M))


def _merge_once(R, kd, vd):
    (k0, v0, k1, v1, k2, v2, k3, v3) = R
    bk, bv = plsc.sort_key_val(kd, vd)
    out = []
    for step, (rk, rv) in enumerate(((k0, v0), (k1, v1), (k2, v2), (k3, v3))):
        rbk = lax.rev(bk, (0,))
        rbv = lax.rev(bv, (0,))
        take = rk <= rbk
        lk = jnp.where(take, rk, rbk)
        lv = jnp.where(take, rv, rbv)
        nk, nv = plsc.sort_key_val(lk, lv)
        if step < 3:
            hk = jnp.where(take, rbk, rk)
            hv = jnp.where(take, rbv, rv)
            bk, bv = plsc.sort_key_val(hk, hv)
        out.extend((nk, nv))
    return tuple(out)


def _select_body(sx_hbm, sy_hbm, sz_hbm, starts_hbm, qs_hbm, map_hbm,
                 sxv, syv, szv, startsv, qv, cd2, cidb, outst):
    wid = lax.axis_index("s") * 2 + lax.axis_index("c")
    iota = _iota16()
    r2 = jnp.float32(R2)

    pltpu.sync_copy(sx_hbm, sxv)
    pltpu.sync_copy(sy_hbm, syv)
    pltpu.sync_copy(sz_hbm, szv)
    pltpu.sync_copy(starts_hbm, startsv)
    pltpu.sync_copy(qs_hbm.at[pl.ds(wid * QPW * 3, QPW * 3)], qv)
    s0 = _sload(startsv, SLOT_S0)

    def per_query(q, _):
        vq = plsc.load_gather(qv, [3 * q + jnp.minimum(iota, 2)])
        qx = vq[0]
        qy = vq[1]
        qz = vq[2]
        cvec = _cell1(vq)
        cx = cvec[0]
        cy = cvec[1]
        cz = cvec[2]
        zlo = jnp.maximum(cz - 1, 0)
        zhi = jnp.minimum(cz + 1, G - 1)

        def over_x(x, cnt):
            def over_y(y, cnt):
                rowc = (x * G + y) * G
                sseg = _sload(startsv, rowc + zlo)
                eseg = _sload(startsv, rowc + zhi + 1)
                nch = (eseg - sseg + 15) // 16

                def chunk(k, cnt):
                    lane = sseg + k * 16 + iota
                    lm = lane < eseg
                    li = jnp.minimum(lane, M - 1)
                    dx = qx - plsc.load_gather(sxv, [li])
                    dy = qy - plsc.load_gather(syv, [li])
                    dz = qz - plsc.load_gather(szv, [li])
                    d2 = (dx * dx + dy * dy) + dz * dz
                    w = lm & (d2 <= r2) & (cnt < CAP - 16)
                    wi = w.astype(_i32)
                    csum = plsc.cumsum(wi)
                    dest = cnt + csum - wi
                    plsc.store_scatter(cd2, [dest], d2, mask=w)
                    plsc.store_scatter(cidb, [dest], lane, mask=w)
                    return cnt + csum[15]

                return lax.fori_loop(0, nch, chunk, cnt)

            return lax.fori_loop(jnp.maximum(cy - 1, 0),
                                 jnp.minimum(cy + 1, G - 1) + 1, over_y, cnt)

        cnt = lax.fori_loop(jnp.maximum(cx - 1, 0),
                            jnp.minimum(cx + 1, G - 1) + 1, over_x,
                            jnp.int32(0))

        inf16 = jnp.full((16,), INF, _f32)
        neg16 = jnp.full((16,), -1, _i32)
        R0 = (inf16, neg16, inf16, neg16, inf16, neg16, inf16, neg16)

        def sel(i, R):
            m = (16 * i + iota) < cnt
            kd = jnp.where(m, cd2[pl.ds(16 * i, 16)], INF)
            vd = jnp.where(m, cidb[pl.ds(16 * i, 16)], -1)
            pred = jnp.min(kd) < jnp.max(R[6])
            return lax.cond(pred, lambda: _merge_once(R, kd, vd), lambda: R)

        R = lax.fori_loop(0, (cnt + 15) // 16, sel, R0)

        base = (q & 31) * 64
        for j in range(4):
            sid = R[2 * j + 1]
            res = jnp.where((sid == s0) | (sid < 0), M, sid)
            outst[pl.ds(base + 16 * j, 16)] = res

        @pl.when((q & 31) == 31)
        def _():
            pltpu.sync_copy(
                outst, map_hbm.at[pl.ds(wid * (QPW * K) + (q - 31) * 64, 2048)])

        return 0

    lax.fori_loop(0, QPW, per_query, 0)


def _run_select(sx, sy, sz, starts, qs_flat):
    mesh = plsc.VectorSubcoreMesh(core_axis_name="c", subcore_axis_name="s",
                                  num_cores=2, num_subcores=16)
    f = pl.kernel(
        _select_body,
        mesh=mesh,
        compiler_params=pltpu.CompilerParams(needs_layout_passes=False, use_tc_tiling_on_sc=False),
        out_type=jax.ShapeDtypeStruct((N * K,), _i32),
        scratch_types=[
            pltpu.VMEM((M,), _f32),
            pltpu.VMEM((M,), _f32),
            pltpu.VMEM((M,), _f32),
            pltpu.VMEM((1024,), _i32),
            pltpu.VMEM((QPW * 3,), _f32),
            pltpu.VMEM((CAP,), _f32),
            pltpu.VMEM((CAP,), _i32),
            pltpu.VMEM((2048,), _i32),
        ],
    )
    return f(sx, sy, sz, starts, qs_flat)


def _gather_body(encT_hbm, spid_hbm, map_hbm, feat_hbm, spidv, idxv, idx2, rows, sem):
    wid = lax.axis_index("s") * 2 + lax.axis_index("c")
    base = wid * (QPW * K)
    iota = _iota16()

    pltpu.sync_copy(spid_hbm, spidv)

    def chunk(ch, _):
        off = base + ch * 2048
        pltpu.sync_copy(map_hbm.at[pl.ds(off, 2048)], idxv)

        def xlat(g, _):
            sid = idxv[pl.ds(16 * g, 16)]
            pid = plsc.load_gather(spidv, [jnp.minimum(sid, M - 1)])
            idx2[pl.ds(16 * g, 16)] = jnp.where(sid == M, M, pid)
            return 0

        lax.fori_loop(0, 128, xlat, 0)

        cps = []
        for j in range(16):
            cps.append(pltpu.async_copy(
                encT_hbm.at[idx2.at[pl.ds(j * 128, 128)]],
                rows.at[pl.ds(j * 128, 128)], sem))
        for cp in cps:
            cp.wait()
        pltpu.sync_copy(rows, feat_hbm.at[pl.ds(off, 2048)])
        return 0

    lax.fori_loop(0, (QPW * K) // 2048, chunk, 0)


def _run_gather(encT, spid, mapv):
    mesh = plsc.VectorSubcoreMesh(core_axis_name="c", subcore_axis_name="s",
                                  num_cores=2, num_subcores=16)
    f = pl.kernel(
        _gather_body,
        mesh=mesh,
        compiler_params=pltpu.CompilerParams(needs_layout_passes=False, use_tc_tiling_on_sc=False),
        out_type=jax.ShapeDtypeStruct((N * K, 8), _f32),
        scratch_types=[
            pltpu.VMEM((M,), _i32),
            pltpu.VMEM((2048,), _i32),
            pltpu.VMEM((2048,), _i32),
            pltpu.VMEM((2048, 8), _f32),
            pltpu.SemaphoreType.DMA,
        ],
    )
    return f(encT, spid, mapv)


BLK_N = 256


def _mlp_body(x_ref, w1_ref, b1_ref, w2_ref, b2_ref, o_ref):
    x = x_ref[...]
    h = jnp.dot(x, w1_ref[...], preferred_element_type=jnp.float32) + b1_ref[...]
    h = jnp.maximum(h, 0.0)
    o_ref[...] = jnp.dot(h, w2_ref[...], preferred_element_type=jnp.float32) + b2_ref[...]


def _mlp(feat, W1p, b1, W2, b2):
    n = feat.shape[0]
    return pl.pallas_call(
        _mlp_body,
        grid=(n // BLK_N,),
        in_specs=[
            pl.BlockSpec((BLK_N, C * K), lambda i: (i, 0)),
            pl.BlockSpec((C * K, BASE), lambda i: (0, 0)),
            pl.BlockSpec((BASE,), lambda i: (0,)),
            pl.BlockSpec((BASE, K), lambda i: (0, 0)),
            pl.BlockSpec((K,), lambda i: (0,)),
        ],
        out_specs=pl.BlockSpec((BLK_N, K), lambda i: (i, 0)),
        out_shape=jax.ShapeDtypeStruct((n, K), jnp.float32),
    )(feat, W1p, b1, W2, b2)


def kernel(encoding_g, volume_mesh_centers, p_grid, W1, b1, W2, b2):
    assert volume_mesh_centers.shape[0] == 1
    pts = p_grid.reshape(M, 3)
    qs_flat = volume_mesh_centers.reshape(N * 3)
    encT = jnp.concatenate(
        [encoding_g.reshape(C, M).T, jnp.zeros((8, C), _f32)], axis=0)
    W1p = W1.reshape(C, K, BASE).transpose(1, 0, 2).reshape(K * C, BASE)

    sx, sy, sz, spid, starts = _run_bin(pts)
    mapv = _run_select(sx, sy, sz, starts, qs_flat)
    feat = _run_gather(encT, spid, mapv)
    out = _mlp(feat.reshape(N, K * C), W1p, b1, W2, b2)
    return out[None]


# speedup vs baseline: 78.2173x; 1.0197x over previous
import functools

import jax
import jax.numpy as jnp
from jax import lax
from jax.experimental import pallas as pl
from jax.experimental.pallas import tpu as pltpu
from jax.experimental.pallas import tpu_sc as plsc

RADIUS = 0.1
K = 64
C = 8
M = 32768
N = 16384
BASE = 512
G = 10
SLOT_S0 = 1016
CAP = 1024
R2 = RADIUS * RADIUS
NW = 32
QPW = N // NW
INF = float("inf")

_i32 = jnp.int32
_f32 = jnp.float32


def _iota16():
    return lax.iota(_i32, 16)


def _cell1(v):
    return jnp.minimum((v * G).astype(_i32), G - 1)


def _splat(x):
    return jnp.full((16,), x, _i32)


def _sload(ref, idx):
    return plsc.load_gather(ref, [_splat(idx)])[0]


def _sstore(ref, idx, val):
    mask0 = _iota16() == 0
    plsc.store_scatter(ref, [_splat(idx)], jnp.full((16,), val), mask=mask0)


def _bin_body(ptsf_hbm, sx_hbm, sy_hbm, sz_hbm, spid_hbm, starts_hbm,
              ptsblkf, cidv, hist, histsh, allhist, startsv, basev,
              destbuf, pidbuf, xbuf, ybuf, zbuf):
    t = lax.axis_index("s")
    npts = M // 16
    p0 = t * npts
    iota = _iota16()

    pltpu.sync_copy(ptsf_hbm.at[pl.ds(3 * p0, 3 * npts)], ptsblkf)

    zeros16 = jnp.zeros((16,), _i32)
    for g in range(64):
        hist[pl.ds(16 * g, 16)] = zeros16

    def p1(g, _):
        i3 = (16 * g + iota) * 3
        px = plsc.load_gather(ptsblkf, [i3])
        py = plsc.load_gather(ptsblkf, [i3 + 1])
        pz = plsc.load_gather(ptsblkf, [i3 + 2])
        cid16 = (_cell1(px) * G + _cell1(py)) * G + _cell1(pz)
        cidv[pl.ds(16 * g, 16)] = cid16
        occ, last = plsc.scan_count(cid16)
        h = plsc.load_gather(hist, [cid16])
        plsc.store_scatter(hist, [cid16], h + occ, mask=last)
        return 0

    lax.fori_loop(0, npts // 16, p1, 0)

    pltpu.sync_copy(hist, histsh.at[t])
    plsc.subcore_barrier()
    pltpu.sync_copy(histsh, allhist)

    def p2(k, carry):
        tot = jnp.zeros((16,), _i32)
        prior = jnp.zeros((16,), _i32)
        for tt in range(16):
            v = allhist[tt, pl.ds(16 * k, 16)]
            prior = jnp.where(tt == t, tot, prior)
            tot = tot + v
        incl = plsc.cumsum(tot)
        excl = incl - tot
        startsv[pl.ds(16 * k, 16)] = excl + carry
        basev[pl.ds(16 * k, 16)] = excl + carry + prior
        return carry + jnp.max(incl)

    lax.fori_loop(0, 64, p2, jnp.int32(0))

    @pl.when(t == 0)
    def _():
        cid0 = cidv[pl.ds(0, 16)][0]
        _sstore(startsv, SLOT_S0, _sload(startsv, cid0))
        pltpu.sync_copy(startsv, starts_hbm)

    def p3(ch, _):
        b = ch * 128

        def serial(g, _):
            i3 = (b + 16 * g + iota) * 3
            xbuf[pl.ds(16 * g, 16)] = plsc.load_gather(ptsblkf, [i3])
            ybuf[pl.ds(16 * g, 16)] = plsc.load_gather(ptsblkf, [i3 + 1])
            zbuf[pl.ds(16 * g, 16)] = plsc.load_gather(ptsblkf, [i3 + 2])
            pidbuf[pl.ds(16 * g, 16)] = p0 + b + 16 * g + iota
            cid16 = cidv[pl.ds(b + 16 * g, 16)]
            occ, last = plsc.scan_count(cid16)
            d = plsc.load_gather(basev, [cid16])
            destbuf[pl.ds(16 * g, 16)] = d + occ - 1
            plsc.store_scatter(basev, [cid16], d + occ, mask=last)
            return 0

        lax.fori_loop(0, 8, serial, 0)

        pltpu.sync_copy(xbuf, sx_hbm.at[destbuf])
        pltpu.sync_copy(ybuf, sy_hbm.at[destbuf])
        pltpu.sync_copy(zbuf, sz_hbm.at[destbuf])
        pltpu.sync_copy(pidbuf, spid_hbm.at[destbuf])
        return 0

    lax.fori_loop(0, 16, p3, 0)


def _run_bin(pts):
    mesh = plsc.VectorSubcoreMesh(core_axis_name="c", subcore_axis_name="s",
                                  num_cores=1, num_subcores=16)
    f = pl.kernel(
        _bin_body,
        mesh=mesh,
        compiler_params=pltpu.CompilerParams(needs_layout_passes=False, use_tc_tiling_on_sc=False),
        out_type=[
            jax.ShapeDtypeStruct((M,), _f32),
            jax.ShapeDtypeStruct((M,), _f32),
            jax.ShapeDtypeStruct((M,), _f32),
            jax.ShapeDtypeStruct((M,), _i32),
            jax.ShapeDtypeStruct((1024,), _i32),
        ],
        scratch_types=[
            pltpu.VMEM((3 * (M // 16),), _f32),
            pltpu.VMEM((M // 16,), _i32),
            pltpu.VMEM((1024,), _i32),
            pltpu.VMEM_SHARED((16, 1024), _i32),
            pltpu.VMEM((16, 1024), _i32),
            pltpu.VMEM((1024,), _i32),
            pltpu.VMEM((1024,), _i32),
            pltpu.VMEM((128,), _i32),
            pltpu.VMEM((128,), _i32),
            pltpu.VMEM((128,), _f32),
            pltpu.VMEM((128,), _f32),
            pltpu.VMEM((128,), _f32),
        ],
    )
    return f(pts.reshape(3 * M))


def _merge_once(R, kd, vd):
    (k0, v0, k1, v1, k2, v2, k3, v3) = R
    bk, bv = plsc.sort_key_val(kd, vd)
    out = []
    for step, (rk, rv) in enumerate(((k0, v0), (k1, v1), (k2, v2), (k3, v3))):
        rbk = lax.rev(bk, (0,))
        rbv = lax.rev(bv, (0,))
        take = rk <= rbk
        lk = jnp.where(take, rk, rbk)
        lv = jnp.where(take, rv, rbv)
        nk, nv = plsc.sort_key_val(lk, lv)
        if step < 3:
            hk = jnp.where(take, rbk, rk)
            hv = jnp.where(take, rbv, rv)
            bk, bv = plsc.sort_key_val(hk, hv)
        out.extend((nk, nv))
    return tuple(out)


def _select_body(sx_hbm, sy_hbm, sz_hbm, starts_hbm, qs_hbm, map_hbm,
                 sxv, syv, szv, startsv, qv, cd2, cidb, outst):
    wid = lax.axis_index("s") * 2 + lax.axis_index("c")
    iota = _iota16()
    r2 = jnp.float32(R2)

    pltpu.sync_copy(sx_hbm, sxv)
    pltpu.sync_copy(sy_hbm, syv)
    pltpu.sync_copy(sz_hbm, szv)
    pltpu.sync_copy(starts_hbm, startsv)
    pltpu.sync_copy(qs_hbm.at[pl.ds(wid * QPW * 3, QPW * 3)], qv)
    s0 = _sload(startsv, SLOT_S0)

    def per_query(q, _):
        vq = plsc.load_gather(qv, [3 * q + jnp.minimum(iota, 2)])
        qx = vq[0]
        qy = vq[1]
        qz = vq[2]
        cvec = _cell1(vq)
        cx = cvec[0]
        cy = cvec[1]
        cz = cvec[2]
        zlo = jnp.maximum(cz - 1, 0)
        zhi = jnp.minimum(cz + 1, G - 1)

        def over_x(x, cnt):
            def over_y(y, cnt):
                rowc = (x * G + y) * G
                sseg = _sload(startsv, rowc + zlo)
                eseg = _sload(startsv, rowc + zhi + 1)
                nch = (eseg - sseg + 15) // 16

                def chunk(k, cnt):
                    lane = sseg + k * 16 + iota
                    lm = lane < eseg
                    li = jnp.minimum(lane, M - 1)
                    dx = qx - plsc.load_gather(sxv, [li])
                    dy = qy - plsc.load_gather(syv, [li])
                    dz = qz - plsc.load_gather(szv, [li])
                    d2 = (dx * dx + dy * dy) + dz * dz
                    w = lm & (d2 <= r2) & (cnt < CAP - 16)
                    wi = w.astype(_i32)
                    csum = plsc.cumsum(wi)
                    dest = cnt + csum - wi
                    plsc.store_scatter(cd2, [dest], d2, mask=w)
                    plsc.store_scatter(cidb, [dest], lane, mask=w)
                    return cnt + csum[15]

                return lax.fori_loop(0, nch, chunk, cnt)

            return lax.fori_loop(jnp.maximum(cy - 1, 0),
                                 jnp.minimum(cy + 1, G - 1) + 1, over_y, cnt)

        cnt = lax.fori_loop(jnp.maximum(cx - 1, 0),
                            jnp.minimum(cx + 1, G - 1) + 1, over_x,
                            jnp.int32(0))

        inf16 = jnp.full((16,), INF, _f32)
        neg16 = jnp.full((16,), -1, _i32)
        R0 = (inf16, neg16, inf16, neg16, inf16, neg16, inf16, neg16)

        def sel(i, R):
            m = (16 * i + iota) < cnt
            kd = jnp.where(m, cd2[pl.ds(16 * i, 16)], INF)
            vd = jnp.where(m, cidb[pl.ds(16 * i, 16)], -1)
            pred = jnp.min(kd) < jnp.max(R[6])
            return lax.cond(pred, lambda: _merge_once(R, kd, vd), lambda: R)

        R = lax.fori_loop(0, (cnt + 15) // 16, sel, R0)

        base = (q & 31) * 64
        for j in range(4):
            sid = R[2 * j + 1]
            res = jnp.where((sid == s0) | (sid < 0), M, sid)
            outst[pl.ds(base + 16 * j, 16)] = res

        @pl.when((q & 31) == 31)
        def _():
            pltpu.sync_copy(
                outst, map_hbm.at[pl.ds(wid * (QPW * K) + (q - 31) * 64, 2048)])

        return 0

    lax.fori_loop(0, QPW, per_query, 0)


def _run_select(sx, sy, sz, starts, qs_flat):
    mesh = plsc.VectorSubcoreMesh(core_axis_name="c", subcore_axis_name="s",
                                  num_cores=2, num_subcores=16)
    f = pl.kernel(
        _select_body,
        mesh=mesh,
        compiler_params=pltpu.CompilerParams(needs_layout_passes=False, use_tc_tiling_on_sc=False),
        out_type=jax.ShapeDtypeStruct((N * K,), _i32),
        scratch_types=[
            pltpu.VMEM((M,), _f32),
            pltpu.VMEM((M,), _f32),
            pltpu.VMEM((M,), _f32),
            pltpu.VMEM((1024,), _i32),
            pltpu.VMEM((QPW * 3,), _f32),
            pltpu.VMEM((CAP,), _f32),
            pltpu.VMEM((CAP,), _i32),
            pltpu.VMEM((2048,), _i32),
        ],
    )
    return f(sx, sy, sz, starts, qs_flat)


def _gather_body(encT_hbm, spid_hbm, map_hbm, feat_hbm, spidv, idxv, idx2, rows, sem):
    wid = lax.axis_index("s") * 2 + lax.axis_index("c")
    base = wid * (QPW * K)
    iota = _iota16()

    pltpu.sync_copy(spid_hbm, spidv)

    def chunk(ch, _):
        off = base + ch * 2048
        pltpu.sync_copy(map_hbm.at[pl.ds(off, 2048)], idxv)

        def xlat(g, _):
            sid = idxv[pl.ds(16 * g, 16)]
            pid = plsc.load_gather(spidv, [jnp.minimum(sid, M - 1)])
            idx2[pl.ds(16 * g, 16)] = jnp.where(sid == M, M, pid)
            return 0

        lax.fori_loop(0, 128, xlat, 0)

        cps = []
        for j in range(16):
            cps.append(pltpu.async_copy(
                encT_hbm.at[idx2.at[pl.ds(j * 128, 128)]],
                rows.at[pl.ds(j * 128, 128)], sem))
        for cp in cps:
            cp.wait()
        pltpu.sync_copy(rows, feat_hbm.at[pl.ds(off, 2048)])
        return 0

    lax.fori_loop(0, (QPW * K) // 2048, chunk, 0)


def _run_gather(encT, spid, mapv):
    mesh = plsc.VectorSubcoreMesh(core_axis_name="c", subcore_axis_name="s",
                                  num_cores=2, num_subcores=16)
    f = pl.kernel(
        _gather_body,
        mesh=mesh,
        compiler_params=pltpu.CompilerParams(needs_layout_passes=False, use_tc_tiling_on_sc=False),
        out_type=jax.ShapeDtypeStruct((N * K, 8), _f32),
        scratch_types=[
            pltpu.VMEM((M,), _i32),
            pltpu.VMEM((2048,), _i32),
            pltpu.VMEM((2048,), _i32),
            pltpu.VMEM((2048, 8), _f32),
            pltpu.SemaphoreType.DMA,
        ],
    )
    return f(encT, spid, mapv)


BLK_N = 256


def _mlp_body(x_ref, w1_ref, b1_ref, w2_ref, b2_ref, o_ref):
    x = x_ref[...]
    h = jnp.dot(x, w1_ref[...], preferred_element_type=jnp.float32) + b1_ref[...]
    h = jnp.maximum(h, 0.0)
    o_ref[...] = jnp.dot(h, w2_ref[...], preferred_element_type=jnp.float32) + b2_ref[...]


def _mlp(feat, W1p, b1, W2, b2):
    n = feat.shape[0]
    return pl.pallas_call(
        _mlp_body,
        grid=(n // BLK_N,),
        in_specs=[
            pl.BlockSpec((BLK_N, C * K), lambda i: (i, 0)),
            pl.BlockSpec((C * K, BASE), lambda i: (0, 0)),
            pl.BlockSpec((BASE,), lambda i: (0,)),
            pl.BlockSpec((BASE, K), lambda i: (0, 0)),
            pl.BlockSpec((K,), lambda i: (0,)),
        ],
        out_specs=pl.BlockSpec((BLK_N, K), lambda i: (i, 0)),
        out_shape=jax.ShapeDtypeStruct((n, K), jnp.float32),
    )(feat, W1p, b1, W2, b2)


def kernel(encoding_g, volume_mesh_centers, p_grid, W1, b1, W2, b2):
    assert volume_mesh_centers.shape[0] == 1
    pts = p_grid.reshape(M, 3)
    qs_flat = volume_mesh_centers.reshape(N * 3)
    encT = jnp.concatenate(
        [encoding_g.reshape(C, M).T, jnp.zeros((8, C), _f32)], axis=0)
    W1p = W1.reshape(C, K, BASE).transpose(1, 0, 2).reshape(K * C, BASE)

    sx, sy, sz, spid, starts = _run_bin(pts)
    mapv = _run_select(sx, sy, sz, starts, qs_flat)
    feat = _run_gather(encT, spid, mapv)
    out = _mlp(feat.reshape(N, K * C), W1p, b1, W2, b2)
    return out[None]


# speedup vs baseline: 82.7302x; 1.0577x over previous
import functools

import jax
import jax.numpy as jnp
from jax import lax
from jax.experimental import pallas as pl
from jax.experimental.pallas import tpu as pltpu
from jax.experimental.pallas import tpu_sc as plsc

RADIUS = 0.1
K = 64
C = 8
M = 32768
N = 16384
BASE = 512
G = 10
SLOT_S0 = 1016
CAP = 1024
R2 = RADIUS * RADIUS
NW = 32
QPW = N // NW
INF = float("inf")

_i32 = jnp.int32
_f32 = jnp.float32


def _iota16():
    return lax.iota(_i32, 16)


def _cell1(v):
    return jnp.minimum((v * G).astype(_i32), G - 1)


def _splat(x):
    return jnp.full((16,), x, _i32)


def _sload(ref, idx):
    return plsc.load_gather(ref, [_splat(idx)])[0]


def _sstore(ref, idx, val):
    mask0 = _iota16() == 0
    plsc.store_scatter(ref, [_splat(idx)], jnp.full((16,), val), mask=mask0)


def _bin_body(ptsf_hbm, sx_hbm, sy_hbm, sz_hbm, spid_hbm, starts_hbm,
              ptsblkf, cidv, hist, histsh, allhist, startsv, basev,
              destbuf, pidbuf, xbuf, ybuf, zbuf):
    t = lax.axis_index("s")
    npts = M // 16
    p0 = t * npts
    iota = _iota16()

    pltpu.sync_copy(ptsf_hbm.at[pl.ds(3 * p0, 3 * npts)], ptsblkf)

    zeros16 = jnp.zeros((16,), _i32)
    for g in range(64):
        hist[pl.ds(16 * g, 16)] = zeros16

    def p1(g, _):
        i3 = (16 * g + iota) * 3
        px = plsc.load_gather(ptsblkf, [i3])
        py = plsc.load_gather(ptsblkf, [i3 + 1])
        pz = plsc.load_gather(ptsblkf, [i3 + 2])
        cid16 = (_cell1(px) * G + _cell1(py)) * G + _cell1(pz)
        cidv[pl.ds(16 * g, 16)] = cid16
        occ, last = plsc.scan_count(cid16)
        h = plsc.load_gather(hist, [cid16])
        plsc.store_scatter(hist, [cid16], h + occ, mask=last)
        return 0

    lax.fori_loop(0, npts // 16, p1, 0)

    pltpu.sync_copy(hist, histsh.at[t])
    plsc.subcore_barrier()
    pltpu.sync_copy(histsh, allhist)

    def p2(k, carry):
        tot = jnp.zeros((16,), _i32)
        prior = jnp.zeros((16,), _i32)
        for tt in range(16):
            v = allhist[tt, pl.ds(16 * k, 16)]
            prior = jnp.where(tt == t, tot, prior)
            tot = tot + v
        incl = plsc.cumsum(tot)
        excl = incl - tot
        startsv[pl.ds(16 * k, 16)] = excl + carry
        basev[pl.ds(16 * k, 16)] = excl + carry + prior
        return carry + jnp.max(incl)

    lax.fori_loop(0, 64, p2, jnp.int32(0))

    @pl.when(t == 0)
    def _():
        cid0 = cidv[pl.ds(0, 16)][0]
        _sstore(startsv, SLOT_S0, _sload(startsv, cid0))
        pltpu.sync_copy(startsv, starts_hbm)

    def p3(ch, _):
        b = ch * 128

        def serial(g, _):
            i3 = (b + 16 * g + iota) * 3
            xbuf[pl.ds(16 * g, 16)] = plsc.load_gather(ptsblkf, [i3])
            ybuf[pl.ds(16 * g, 16)] = plsc.load_gather(ptsblkf, [i3 + 1])
            zbuf[pl.ds(16 * g, 16)] = plsc.load_gather(ptsblkf, [i3 + 2])
            pidbuf[pl.ds(16 * g, 16)] = p0 + b + 16 * g + iota
            cid16 = cidv[pl.ds(b + 16 * g, 16)]
            occ, last = plsc.scan_count(cid16)
            d = plsc.load_gather(basev, [cid16])
            destbuf[pl.ds(16 * g, 16)] = d + occ - 1
            plsc.store_scatter(basev, [cid16], d + occ, mask=last)
            return 0

        lax.fori_loop(0, 8, serial, 0)

        pltpu.sync_copy(xbuf, sx_hbm.at[destbuf])
        pltpu.sync_copy(ybuf, sy_hbm.at[destbuf])
        pltpu.sync_copy(zbuf, sz_hbm.at[destbuf])
        pltpu.sync_copy(pidbuf, spid_hbm.at[destbuf])
        return 0

    lax.fori_loop(0, 16, p3, 0)


def _run_bin(pts):
    mesh = plsc.VectorSubcoreMesh(core_axis_name="c", subcore_axis_name="s",
                                  num_cores=1, num_subcores=16)
    f = pl.kernel(
        _bin_body,
        mesh=mesh,
        compiler_params=pltpu.CompilerParams(needs_layout_passes=False, use_tc_tiling_on_sc=False),
        out_type=[
            jax.ShapeDtypeStruct((M,), _f32),
            jax.ShapeDtypeStruct((M,), _f32),
            jax.ShapeDtypeStruct((M,), _f32),
            jax.ShapeDtypeStruct((M,), _i32),
            jax.ShapeDtypeStruct((1024,), _i32),
        ],
        scratch_types=[
            pltpu.VMEM((3 * (M // 16),), _f32),
            pltpu.VMEM((M // 16,), _i32),
            pltpu.VMEM((1024,), _i32),
            pltpu.VMEM_SHARED((16, 1024), _i32),
            pltpu.VMEM((16, 1024), _i32),
            pltpu.VMEM((1024,), _i32),
            pltpu.VMEM((1024,), _i32),
            pltpu.VMEM((128,), _i32),
            pltpu.VMEM((128,), _i32),
            pltpu.VMEM((128,), _f32),
            pltpu.VMEM((128,), _f32),
            pltpu.VMEM((128,), _f32),
        ],
    )
    return f(pts.reshape(3 * M))


def _merge_once(R, kd, vd):
    (k0, v0, k1, v1, k2, v2, k3, v3) = R
    bk, bv = plsc.sort_key_val(kd, vd)
    out = []
    for step, (rk, rv) in enumerate(((k0, v0), (k1, v1), (k2, v2), (k3, v3))):
        rbk = lax.rev(bk, (0,))
        rbv = lax.rev(bv, (0,))
        take = rk <= rbk
        lk = jnp.where(take, rk, rbk)
        lv = jnp.where(take, rv, rbv)
        nk, nv = plsc.sort_key_val(lk, lv)
        if step < 3:
            hk = jnp.where(take, rbk, rk)
            hv = jnp.where(take, rbv, rv)
            bk, bv = plsc.sort_key_val(hk, hv)
        out.extend((nk, nv))
    return tuple(out)


def _select_body(sx_hbm, sy_hbm, sz_hbm, starts_hbm, qs_hbm, map_hbm,
                 sxv, syv, szv, startsv, qv, cd2, cidb, outst):
    wid = lax.axis_index("s") * 2 + lax.axis_index("c")
    iota = _iota16()
    r2 = jnp.float32(R2)

    pltpu.sync_copy(sx_hbm, sxv)
    pltpu.sync_copy(sy_hbm, syv)
    pltpu.sync_copy(sz_hbm, szv)
    pltpu.sync_copy(starts_hbm, startsv)
    pltpu.sync_copy(qs_hbm.at[pl.ds(wid * QPW * 3, QPW * 3)], qv)
    s0 = _sload(startsv, SLOT_S0)

    def per_query(q, _):
        vq = plsc.load_gather(qv, [3 * q + jnp.minimum(iota, 2)])
        qx = vq[0]
        qy = vq[1]
        qz = vq[2]
        cvec = _cell1(vq)
        cx = cvec[0]
        cy = cvec[1]
        cz = cvec[2]
        zlo = jnp.maximum(cz - 1, 0)
        zhi = jnp.minimum(cz + 1, G - 1)

        def over_x(x, cnt):
            def over_y(y, cnt):
                rowc = (x * G + y) * G
                sseg = _sload(startsv, rowc + zlo)
                eseg = _sload(startsv, rowc + zhi + 1)
                nch = (eseg - sseg + 15) // 16

                def chunk2(k, cnt):
                    guard = cnt < CAP - 32
                    res = []
                    for u in range(2):
                        lane = sseg + (2 * k + u) * 16 + iota
                        lm = lane < eseg
                        li = jnp.minimum(lane, M - 1)
                        dx = qx - plsc.load_gather(sxv, [li])
                        dy = qy - plsc.load_gather(syv, [li])
                        dz = qz - plsc.load_gather(szv, [li])
                        d2 = (dx * dx + dy * dy) + dz * dz
                        w = lm & (d2 <= r2) & guard
                        wi = w.astype(_i32)
                        csum = plsc.cumsum(wi)
                        res.append((lane, d2, w, wi, csum))
                    lane0, d20, w0, wi0, csum0 = res[0]
                    lane1, d21, w1, wi1, csum1 = res[1]
                    dest0 = cnt + csum0 - wi0
                    plsc.store_scatter(cd2, [dest0], d20, mask=w0)
                    plsc.store_scatter(cidb, [dest0], lane0, mask=w0)
                    cnt1 = cnt + csum0[15]
                    dest1 = cnt1 + csum1 - wi1
                    plsc.store_scatter(cd2, [dest1], d21, mask=w1)
                    plsc.store_scatter(cidb, [dest1], lane1, mask=w1)
                    return cnt1 + csum1[15]

                return lax.fori_loop(0, (nch + 1) // 2, chunk2, cnt)

            return lax.fori_loop(jnp.maximum(cy - 1, 0),
                                 jnp.minimum(cy + 1, G - 1) + 1, over_y, cnt)

        cnt = lax.fori_loop(jnp.maximum(cx - 1, 0),
                            jnp.minimum(cx + 1, G - 1) + 1, over_x,
                            jnp.int32(0))

        inf16 = jnp.full((16,), INF, _f32)
        neg16 = jnp.full((16,), -1, _i32)
        R0 = (inf16, neg16, inf16, neg16, inf16, neg16, inf16, neg16)

        def sel(i, R):
            m = (16 * i + iota) < cnt
            kd = jnp.where(m, cd2[pl.ds(16 * i, 16)], INF)
            vd = jnp.where(m, cidb[pl.ds(16 * i, 16)], -1)
            pred = jnp.min(kd) < jnp.max(R[6])
            return lax.cond(pred, lambda: _merge_once(R, kd, vd), lambda: R)

        R = lax.fori_loop(0, (cnt + 15) // 16, sel, R0)

        base = (q & 31) * 64
        for j in range(4):
            sid = R[2 * j + 1]
            res = jnp.where((sid == s0) | (sid < 0), M, sid)
            outst[pl.ds(base + 16 * j, 16)] = res

        @pl.when((q & 31) == 31)
        def _():
            pltpu.sync_copy(
                outst, map_hbm.at[pl.ds(wid * (QPW * K) + (q - 31) * 64, 2048)])

        return 0

    lax.fori_loop(0, QPW, per_query, 0)


def _run_select(sx, sy, sz, starts, qs_flat):
    mesh = plsc.VectorSubcoreMesh(core_axis_name="c", subcore_axis_name="s",
                                  num_cores=2, num_subcores=16)
    f = pl.kernel(
        _select_body,
        mesh=mesh,
        compiler_params=pltpu.CompilerParams(needs_layout_passes=False, use_tc_tiling_on_sc=False),
        out_type=jax.ShapeDtypeStruct((N * K,), _i32),
        scratch_types=[
            pltpu.VMEM((M,), _f32),
            pltpu.VMEM((M,), _f32),
            pltpu.VMEM((M,), _f32),
            pltpu.VMEM((1024,), _i32),
            pltpu.VMEM((QPW * 3,), _f32),
            pltpu.VMEM((CAP,), _f32),
            pltpu.VMEM((CAP,), _i32),
            pltpu.VMEM((2048,), _i32),
        ],
    )
    return f(sx, sy, sz, starts, qs_flat)


def _gather_body(encT_hbm, spid_hbm, map_hbm, feat_hbm, spidv, idxv, idx2, rows, sem):
    wid = lax.axis_index("s") * 2 + lax.axis_index("c")
    base = wid * (QPW * K)
    iota = _iota16()

    pltpu.sync_copy(spid_hbm, spidv)

    def chunk(ch, _):
        off = base + ch * 2048
        pltpu.sync_copy(map_hbm.at[pl.ds(off, 2048)], idxv)

        def xlat(g, _):
            sid = idxv[pl.ds(16 * g, 16)]
            pid = plsc.load_gather(spidv, [jnp.minimum(sid, M - 1)])
            idx2[pl.ds(16 * g, 16)] = jnp.where(sid == M, M, pid)
            return 0

        lax.fori_loop(0, 128, xlat, 0)

        cps = []
        for j in range(16):
            cps.append(pltpu.async_copy(
                encT_hbm.at[idx2.at[pl.ds(j * 128, 128)]],
                rows.at[pl.ds(j * 128, 128)], sem))
        for cp in cps:
            cp.wait()
        pltpu.sync_copy(rows, feat_hbm.at[pl.ds(off, 2048)])
        return 0

    lax.fori_loop(0, (QPW * K) // 2048, chunk, 0)


def _run_gather(encT, spid, mapv):
    mesh = plsc.VectorSubcoreMesh(core_axis_name="c", subcore_axis_name="s",
                                  num_cores=2, num_subcores=16)
    f = pl.kernel(
        _gather_body,
        mesh=mesh,
        compiler_params=pltpu.CompilerParams(needs_layout_passes=False, use_tc_tiling_on_sc=False),
        out_type=jax.ShapeDtypeStruct((N * K, 8), _f32),
        scratch_types=[
            pltpu.VMEM((M,), _i32),
            pltpu.VMEM((2048,), _i32),
            pltpu.VMEM((2048,), _i32),
            pltpu.VMEM((2048, 8), _f32),
            pltpu.SemaphoreType.DMA,
        ],
    )
    return f(encT, spid, mapv)


BLK_N = 256


def _mlp_body(x_ref, w1_ref, b1_ref, w2_ref, b2_ref, o_ref):
    x = x_ref[...]
    h = jnp.dot(x, w1_ref[...], preferred_element_type=jnp.float32) + b1_ref[...]
    h = jnp.maximum(h, 0.0)
    o_ref[...] = jnp.dot(h, w2_ref[...], preferred_element_type=jnp.float32) + b2_ref[...]


def _mlp(feat, W1p, b1, W2, b2):
    n = feat.shape[0]
    return pl.pallas_call(
        _mlp_body,
        grid=(n // BLK_N,),
        in_specs=[
            pl.BlockSpec((BLK_N, C * K), lambda i: (i, 0)),
            pl.BlockSpec((C * K, BASE), lambda i: (0, 0)),
            pl.BlockSpec((BASE,), lambda i: (0,)),
            pl.BlockSpec((BASE, K), lambda i: (0, 0)),
            pl.BlockSpec((K,), lambda i: (0,)),
        ],
        out_specs=pl.BlockSpec((BLK_N, K), lambda i: (i, 0)),
        out_shape=jax.ShapeDtypeStruct((n, K), jnp.float32),
    )(feat, W1p, b1, W2, b2)


def kernel(encoding_g, volume_mesh_centers, p_grid, W1, b1, W2, b2):
    assert volume_mesh_centers.shape[0] == 1
    pts = p_grid.reshape(M, 3)
    qs_flat = volume_mesh_centers.reshape(N * 3)
    encT = jnp.concatenate(
        [encoding_g.reshape(C, M).T, jnp.zeros((8, C), _f32)], axis=0)
    W1p = W1.reshape(C, K, BASE).transpose(1, 0, 2).reshape(K * C, BASE)

    sx, sy, sz, spid, starts = _run_bin(pts)
    mapv = _run_select(sx, sy, sz, starts, qs_flat)
    feat = _run_gather(encT, spid, mapv)
    out = _mlp(feat.reshape(N, K * C), W1p, b1, W2, b2)
    return out[None]


# speedup vs baseline: 87.0551x; 1.0523x over previous
import functools

import jax
import jax.numpy as jnp
from jax import lax
from jax.experimental import pallas as pl
from jax.experimental.pallas import tpu as pltpu
from jax.experimental.pallas import tpu_sc as plsc

RADIUS = 0.1
K = 64
C = 8
M = 32768
N = 16384
BASE = 512
G = 10
GZ = 30
NCELL = G * G * GZ
NSTART = 3072
SLOT_S0 = 3064
CAP = 1024
R2 = RADIUS * RADIUS
NW = 32
QPW = N // NW
INF = float("inf")

_i32 = jnp.int32
_f32 = jnp.float32


def _iota16():
    return lax.iota(_i32, 16)


def _cell1(v):
    return jnp.minimum((v * G).astype(_i32), G - 1)


def _cellz(v):
    return jnp.minimum((v * GZ).astype(_i32), GZ - 1)


def _splat(x):
    return jnp.full((16,), x, _i32)


def _sload(ref, idx):
    return plsc.load_gather(ref, [_splat(idx)])[0]


def _sstore(ref, idx, val):
    mask0 = _iota16() == 0
    plsc.store_scatter(ref, [_splat(idx)], jnp.full((16,), val), mask=mask0)


def _bin_body(ptsf_hbm, sx_hbm, sy_hbm, sz_hbm, spid_hbm, starts_hbm,
              ptsblkf, cidv, hist, histsh, allhist, startsv, basev,
              destbuf, pidbuf, xbuf, ybuf, zbuf):
    t = lax.axis_index("s")
    npts = M // 16
    p0 = t * npts
    iota = _iota16()

    pltpu.sync_copy(ptsf_hbm.at[pl.ds(3 * p0, 3 * npts)], ptsblkf)

    zeros16 = jnp.zeros((16,), _i32)
    for g in range(NSTART // 16):
        hist[pl.ds(16 * g, 16)] = zeros16

    def p1(g, _):
        i3 = (16 * g + iota) * 3
        px = plsc.load_gather(ptsblkf, [i3])
        py = plsc.load_gather(ptsblkf, [i3 + 1])
        pz = plsc.load_gather(ptsblkf, [i3 + 2])
        cid16 = (_cell1(px) * G + _cell1(py)) * GZ + _cellz(pz)
        cidv[pl.ds(16 * g, 16)] = cid16
        occ, last = plsc.scan_count(cid16)
        h = plsc.load_gather(hist, [cid16])
        plsc.store_scatter(hist, [cid16], h + occ, mask=last)
        return 0

    lax.fori_loop(0, npts // 16, p1, 0)

    pltpu.sync_copy(hist, histsh.at[t])
    plsc.subcore_barrier()
    pltpu.sync_copy(histsh, allhist)

    def p2(k, carry):
        tot = jnp.zeros((16,), _i32)
        prior = jnp.zeros((16,), _i32)
        for tt in range(16):
            v = allhist[tt, pl.ds(16 * k, 16)]
            prior = jnp.where(tt == t, tot, prior)
            tot = tot + v
        incl = plsc.cumsum(tot)
        excl = incl - tot
        startsv[pl.ds(16 * k, 16)] = excl + carry
        basev[pl.ds(16 * k, 16)] = excl + carry + prior
        return carry + jnp.max(incl)

    lax.fori_loop(0, NSTART // 16, p2, jnp.int32(0))

    @pl.when(t == 0)
    def _():
        cid0 = cidv[pl.ds(0, 16)][0]
        _sstore(startsv, SLOT_S0, _sload(startsv, cid0))
        pltpu.sync_copy(startsv, starts_hbm)

    def p3(ch, _):
        b = ch * 128

        def serial(g, _):
            i3 = (b + 16 * g + iota) * 3
            xbuf[pl.ds(16 * g, 16)] = plsc.load_gather(ptsblkf, [i3])
            ybuf[pl.ds(16 * g, 16)] = plsc.load_gather(ptsblkf, [i3 + 1])
            zbuf[pl.ds(16 * g, 16)] = plsc.load_gather(ptsblkf, [i3 + 2])
            pidbuf[pl.ds(16 * g, 16)] = p0 + b + 16 * g + iota
            cid16 = cidv[pl.ds(b + 16 * g, 16)]
            occ, last = plsc.scan_count(cid16)
            d = plsc.load_gather(basev, [cid16])
            destbuf[pl.ds(16 * g, 16)] = d + occ - 1
            plsc.store_scatter(basev, [cid16], d + occ, mask=last)
            return 0

        lax.fori_loop(0, 8, serial, 0)

        pltpu.sync_copy(xbuf, sx_hbm.at[destbuf])
        pltpu.sync_copy(ybuf, sy_hbm.at[destbuf])
        pltpu.sync_copy(zbuf, sz_hbm.at[destbuf])
        pltpu.sync_copy(pidbuf, spid_hbm.at[destbuf])
        return 0

    lax.fori_loop(0, 16, p3, 0)


def _run_bin(pts):
    mesh = plsc.VectorSubcoreMesh(core_axis_name="c", subcore_axis_name="s",
                                  num_cores=1, num_subcores=16)
    f = pl.kernel(
        _bin_body,
        mesh=mesh,
        compiler_params=pltpu.CompilerParams(needs_layout_passes=False, use_tc_tiling_on_sc=False),
        out_type=[
            jax.ShapeDtypeStruct((M,), _f32),
            jax.ShapeDtypeStruct((M,), _f32),
            jax.ShapeDtypeStruct((M,), _f32),
            jax.ShapeDtypeStruct((M,), _i32),
            jax.ShapeDtypeStruct((NSTART,), _i32),
        ],
        scratch_types=[
            pltpu.VMEM((3 * (M // 16),), _f32),
            pltpu.VMEM((M // 16,), _i32),
            pltpu.VMEM((NSTART,), _i32),
            pltpu.VMEM_SHARED((16, NSTART), _i32),
            pltpu.VMEM((16, NSTART), _i32),
            pltpu.VMEM((NSTART,), _i32),
            pltpu.VMEM((NSTART,), _i32),
            pltpu.VMEM((128,), _i32),
            pltpu.VMEM((128,), _i32),
            pltpu.VMEM((128,), _f32),
            pltpu.VMEM((128,), _f32),
            pltpu.VMEM((128,), _f32),
        ],
    )
    return f(pts.reshape(3 * M))


def _merge_once(R, kd, vd):
    (k0, v0, k1, v1, k2, v2, k3, v3) = R
    bk, bv = plsc.sort_key_val(kd, vd)
    out = []
    for step, (rk, rv) in enumerate(((k0, v0), (k1, v1), (k2, v2), (k3, v3))):
        rbk = lax.rev(bk, (0,))
        rbv = lax.rev(bv, (0,))
        take = rk <= rbk
        lk = jnp.where(take, rk, rbk)
        lv = jnp.where(take, rv, rbv)
        nk, nv = plsc.sort_key_val(lk, lv)
        if step < 3:
            hk = jnp.where(take, rbk, rk)
            hv = jnp.where(take, rbv, rv)
            bk, bv = plsc.sort_key_val(hk, hv)
        out.extend((nk, nv))
    return tuple(out)


def _select_body(sx_hbm, sy_hbm, sz_hbm, starts_hbm, qs_hbm, map_hbm,
                 sxv, syv, szv, startsv, qv, cd2, cidb, outst):
    wid = lax.axis_index("s") * 2 + lax.axis_index("c")
    iota = _iota16()
    r2 = jnp.float32(R2)

    pltpu.sync_copy(sx_hbm, sxv)
    pltpu.sync_copy(sy_hbm, syv)
    pltpu.sync_copy(sz_hbm, szv)
    pltpu.sync_copy(starts_hbm, startsv)
    pltpu.sync_copy(qs_hbm.at[pl.ds(wid * QPW * 3, QPW * 3)], qv)
    s0 = _sload(startsv, SLOT_S0)

    def per_query(q, _):
        vq = plsc.load_gather(qv, [3 * q + jnp.minimum(iota, 2)])
        qx = vq[0]
        qy = vq[1]
        qz = vq[2]
        cvec = _cell1(vq)
        zvec = _cellz(vq)
        cx = cvec[0]
        cy = cvec[1]
        cz = zvec[2]
        zlo = jnp.maximum(cz - 3, 0)
        zhi = jnp.minimum(cz + 3, GZ - 1)

        def over_x(x, cnt):
            def over_y(y, cnt):
                rowc = (x * G + y) * GZ
                sseg = _sload(startsv, rowc + zlo)
                eseg = _sload(startsv, rowc + zhi + 1)
                nch = (eseg - sseg + 15) // 16

                def chunk2(k, cnt):
                    guard = cnt < CAP - 32
                    res = []
                    for u in range(2):
                        lane = sseg + (2 * k + u) * 16 + iota
                        lm = lane < eseg
                        li = jnp.minimum(lane, M - 1)
                        dx = qx - plsc.load_gather(sxv, [li])
                        dy = qy - plsc.load_gather(syv, [li])
                        dz = qz - plsc.load_gather(szv, [li])
                        d2 = (dx * dx + dy * dy) + dz * dz
                        w = lm & (d2 <= r2) & guard
                        wi = w.astype(_i32)
                        csum = plsc.cumsum(wi)
                        res.append((lane, d2, w, wi, csum))
                    lane0, d20, w0, wi0, csum0 = res[0]
                    lane1, d21, w1, wi1, csum1 = res[1]
                    dest0 = cnt + csum0 - wi0
                    plsc.store_scatter(cd2, [dest0], d20, mask=w0)
                    plsc.store_scatter(cidb, [dest0], lane0, mask=w0)
                    cnt1 = cnt + csum0[15]
                    dest1 = cnt1 + csum1 - wi1
                    plsc.store_scatter(cd2, [dest1], d21, mask=w1)
                    plsc.store_scatter(cidb, [dest1], lane1, mask=w1)
                    return cnt1 + csum1[15]

                return lax.fori_loop(0, (nch + 1) // 2, chunk2, cnt)

            return lax.fori_loop(jnp.maximum(cy - 1, 0),
                                 jnp.minimum(cy + 1, G - 1) + 1, over_y, cnt)

        cnt = lax.fori_loop(jnp.maximum(cx - 1, 0),
                            jnp.minimum(cx + 1, G - 1) + 1, over_x,
                            jnp.int32(0))

        inf16 = jnp.full((16,), INF, _f32)
        neg16 = jnp.full((16,), -1, _i32)
        R0 = (inf16, neg16, inf16, neg16, inf16, neg16, inf16, neg16)

        def sel(i, R):
            m = (16 * i + iota) < cnt
            kd = jnp.where(m, cd2[pl.ds(16 * i, 16)], INF)
            vd = jnp.where(m, cidb[pl.ds(16 * i, 16)], -1)
            pred = jnp.min(kd) < jnp.max(R[6])
            return lax.cond(pred, lambda: _merge_once(R, kd, vd), lambda: R)

        R = lax.fori_loop(0, (cnt + 15) // 16, sel, R0)

        base = (q & 31) * 64
        for j in range(4):
            sid = R[2 * j + 1]
            res = jnp.where((sid == s0) | (sid < 0), M, sid)
            outst[pl.ds(base + 16 * j, 16)] = res

        @pl.when((q & 31) == 31)
        def _():
            pltpu.sync_copy(
                outst, map_hbm.at[pl.ds(wid * (QPW * K) + (q - 31) * 64, 2048)])

        return 0

    lax.fori_loop(0, QPW, per_query, 0)


def _run_select(sx, sy, sz, starts, qs_flat):
    mesh = plsc.VectorSubcoreMesh(core_axis_name="c", subcore_axis_name="s",
                                  num_cores=2, num_subcores=16)
    f = pl.kernel(
        _select_body,
        mesh=mesh,
        compiler_params=pltpu.CompilerParams(needs_layout_passes=False, use_tc_tiling_on_sc=False),
        out_type=jax.ShapeDtypeStruct((N * K,), _i32),
        scratch_types=[
            pltpu.VMEM((M,), _f32),
            pltpu.VMEM((M,), _f32),
            pltpu.VMEM((M,), _f32),
            pltpu.VMEM((NSTART,), _i32),
            pltpu.VMEM((QPW * 3,), _f32),
            pltpu.VMEM((CAP,), _f32),
            pltpu.VMEM((CAP,), _i32),
            pltpu.VMEM((2048,), _i32),
        ],
    )
    return f(sx, sy, sz, starts, qs_flat)


def _gather_body(encT_hbm, spid_hbm, map_hbm, feat_hbm, spidv, idxv, idx2, rows, sem):
    wid = lax.axis_index("s") * 2 + lax.axis_index("c")
    base = wid * (QPW * K)
    iota = _iota16()

    pltpu.sync_copy(spid_hbm, spidv)

    def chunk(ch, _):
        off = base + ch * 2048
        pltpu.sync_copy(map_hbm.at[pl.ds(off, 2048)], idxv)

        def xlat(g, _):
            sid = idxv[pl.ds(16 * g, 16)]
            pid = plsc.load_gather(spidv, [jnp.minimum(sid, M - 1)])
            idx2[pl.ds(16 * g, 16)] = jnp.where(sid == M, M, pid)
            return 0

        lax.fori_loop(0, 128, xlat, 0)

        cps = []
        for j in range(16):
            cps.append(pltpu.async_copy(
                encT_hbm.at[idx2.at[pl.ds(j * 128, 128)]],
                rows.at[pl.ds(j * 128, 128)], sem))
        for cp in cps:
            cp.wait()
        pltpu.sync_copy(rows, feat_hbm.at[pl.ds(off, 2048)])
        return 0

    lax.fori_loop(0, (QPW * K) // 2048, chunk, 0)


def _run_gather(encT, spid, mapv):
    mesh = plsc.VectorSubcoreMesh(core_axis_name="c", subcore_axis_name="s",
                                  num_cores=2, num_subcores=16)
    f = pl.kernel(
        _gather_body,
        mesh=mesh,
        compiler_params=pltpu.CompilerParams(needs_layout_passes=False, use_tc_tiling_on_sc=False),
        out_type=jax.ShapeDtypeStruct((N * K, 8), _f32),
        scratch_types=[
            pltpu.VMEM((M,), _i32),
            pltpu.VMEM((2048,), _i32),
            pltpu.VMEM((2048,), _i32),
            pltpu.VMEM((2048, 8), _f32),
            pltpu.SemaphoreType.DMA,
        ],
    )
    return f(encT, spid, mapv)


BLK_N = 256


def _mlp_body(x_ref, w1_ref, b1_ref, w2_ref, b2_ref, o_ref):
    x = x_ref[...]
    h = jnp.dot(x, w1_ref[...], preferred_element_type=jnp.float32) + b1_ref[...]
    h = jnp.maximum(h, 0.0)
    o_ref[...] = jnp.dot(h, w2_ref[...], preferred_element_type=jnp.float32) + b2_ref[...]


def _mlp(feat, W1p, b1, W2, b2):
    n = feat.shape[0]
    return pl.pallas_call(
        _mlp_body,
        grid=(n // BLK_N,),
        in_specs=[
            pl.BlockSpec((BLK_N, C * K), lambda i: (i, 0)),
            pl.BlockSpec((C * K, BASE), lambda i: (0, 0)),
            pl.BlockSpec((BASE,), lambda i: (0,)),
            pl.BlockSpec((BASE, K), lambda i: (0, 0)),
            pl.BlockSpec((K,), lambda i: (0,)),
        ],
        out_specs=pl.BlockSpec((BLK_N, K), lambda i: (i, 0)),
        out_shape=jax.ShapeDtypeStruct((n, K), jnp.float32),
    )(feat, W1p, b1, W2, b2)


def kernel(encoding_g, volume_mesh_centers, p_grid, W1, b1, W2, b2):
    assert volume_mesh_centers.shape[0] == 1
    pts = p_grid.reshape(M, 3)
    qs_flat = volume_mesh_centers.reshape(N * 3)
    encT = jnp.concatenate(
        [encoding_g.reshape(C, M).T, jnp.zeros((8, C), _f32)], axis=0)
    W1p = W1.reshape(C, K, BASE).transpose(1, 0, 2).reshape(K * C, BASE)

    sx, sy, sz, spid, starts = _run_bin(pts)
    mapv = _run_select(sx, sy, sz, starts, qs_flat)
    feat = _run_gather(encT, spid, mapv)
    out = _mlp(feat.reshape(N, K * C), W1p, b1, W2, b2)
    return out[None]
